# baseline (speedup 1.0000x reference)
import functools

import jax
import jax.numpy as jnp
from jax import lax
from jax.experimental import pallas as pl
from jax.experimental.pallas import tpu as pltpu
from jax.experimental.pallas import tpu_sc as plsc

NUM_WORKERS = 32
CHUNK = 128
NBUF = 5


def _make_gather(n_chunks, d):
    mesh = plsc.VectorSubcoreMesh(core_axis_name="c", subcore_axis_name="s")
    half = CHUNK // 2
    total_chunks = NUM_WORKERS * n_chunks

    @functools.partial(
        pl.kernel,
        out_type=jax.ShapeDtypeStruct((total_chunks * half, 2 * d),
                                      jnp.float32),
        mesh=mesh,
        scratch_types=(
            [pltpu.VMEM((n_chunks, half), jnp.int32),
             pltpu.VMEM((n_chunks, half), jnp.int32),
             pltpu.VMEM((NBUF, CHUNK, d), jnp.float32)]
            + [pltpu.SemaphoreType.DMA] * (2 * NBUF)
        ),
        compiler_params=pltpu.CompilerParams(use_tc_tiling_on_sc=False),
    )
    def gather_kernel(idxe_hbm, idxo_hbm, table_hbm, out_hbm,
                      idxe_v, idxo_v, rows_v, *sems):
        gsem = sems[:NBUF]
        wsem = sems[NBUF:]
        wid = lax.axis_index("s") * 2 + lax.axis_index("c")
        pltpu.sync_copy(idxe_hbm.at[wid], idxe_v)
        pltpu.sync_copy(idxo_hbm.at[wid], idxo_v)

        dummy_full = table_hbm.at[pl.ds(0, CHUNK)]

        def fire(j, b):
            pltpu.async_copy(table_hbm.at[idxe_v.at[j]],
                             rows_v.at[b, pl.ds(0, half)], gsem[b])
            pltpu.async_copy(table_hbm.at[idxo_v.at[j]],
                             rows_v.at[b, pl.ds(half, half)], gsem[b])

        def drain_g(b):
            pltpu.make_async_copy(dummy_full, rows_v.at[b], gsem[b]).wait()

        def put(j, b):
            q = (wid * n_chunks + j) * half
            pltpu.async_copy(rows_v.at[b, pl.ds(0, half)],
                             out_hbm.at[pl.ds(q, half), pl.ds(0, d)],
                             wsem[b])
            pltpu.async_copy(rows_v.at[b, pl.ds(half, half)],
                             out_hbm.at[pl.ds(q, half), pl.ds(d, d)],
                             wsem[b])

        def drain_w(b):
            pltpu.make_async_copy(dummy_full, rows_v.at[b], wsem[b]).wait()

        for b in range(NBUF - 1):
            fire(b, b)

        assert n_chunks % NBUF == 0

        def outer(g0, carry):
            for i in range(NBUF):
                j = g0 * NBUF + i
                fb = (i + NBUF - 1) % NBUF

                @pl.when(j >= 1)
                def _():
                    drain_w(fb)

                @pl.when(j + NBUF - 1 < n_chunks)
                def _():
                    fire(j + NBUF - 1, fb)

                drain_g(i)
                put(j, i)
            return carry

        lax.fori_loop(0, n_chunks // NBUF, outer, 0, unroll=False)
        drain_w((n_chunks - 1) % NBUF)

    return gather_kernel


def kernel(code, embedding_table):
    b, s = code.shape
    v, d = embedding_table.shape
    total = b * s
    assert total % (NUM_WORKERS * CHUNK) == 0
    n_chunks = total // (NUM_WORKERS * CHUNK)
    half = CHUNK // 2
    flat = code.reshape(total).astype(jnp.int32)
    idxe = flat[0::2].reshape(NUM_WORKERS, n_chunks, half)
    idxo = flat[1::2].reshape(NUM_WORKERS, n_chunks, half)
    out = _make_gather(n_chunks, d)(idxe, idxo, embedding_table)
    return out.reshape(b, s, d)
